# Initial kernel scaffold; baseline (speedup 1.0000x reference)
import jax
import jax.numpy as jnp
from jax.experimental import pallas as pl


def kernel(x, edge_index, batch, W0, b0, W1, b1, Wf, bf):
    raise NotImplementedError("write your pallas kernel here")


# speedup vs baseline: 17.6214x; 17.6214x over previous
import functools

import jax
import jax.numpy as jnp
from jax import lax
from jax.experimental import pallas as pl
from jax.experimental.pallas import tpu as pltpu
from jax.experimental.pallas import tpu_sc as plsc

NC = 2
NS = 16
NW = NC * NS


def _make_edge_pass(n, c, nch, k):
    mesh = plsc.VectorSubcoreMesh(
        core_axis_name="c", subcore_axis_name="s",
        num_cores=NC, num_subcores=NS)
    rpt = n // NS

    @functools.partial(
        pl.kernel,
        out_type=jax.ShapeDtypeStruct((NC, n, c), jnp.float32),
        mesh=mesh,
        scratch_types=[
            pltpu.VMEM((nch, k), jnp.int32),
            pltpu.VMEM((nch, k), jnp.int32),
            pltpu.VMEM((k, c), jnp.float32),
            pltpu.VMEM_SHARED((n, c), jnp.float32),
            pltpu.SemaphoreType.DMA,
        ],
        compiler_params=pltpu.CompilerParams(use_tc_tiling_on_sc=False),
    )
    def edge_pass(table_hbm, src_hbm, dst_hbm, zeros_hbm, part_hbm,
                  src_v, dst_v, rows_v, acc_sh, sem):
        cid = lax.axis_index("c")
        sid = lax.axis_index("s")
        wid = sid * NC + cid
        r0 = sid * rpt
        pltpu.sync_copy(zeros_hbm.at[pl.ds(r0, rpt)], acc_sh.at[pl.ds(r0, rpt)])
        pltpu.sync_copy(src_hbm.at[wid], src_v)
        pltpu.sync_copy(dst_hbm.at[wid], dst_v)
        plsc.subcore_barrier()

        def body(j, carry):
            pltpu.async_copy(table_hbm.at[src_v.at[j]], rows_v, sem).wait()
            pltpu.sync_copy(rows_v, acc_sh.at[dst_v.at[j]], add=True)
            return carry

        lax.fori_loop(0, nch, body, 0)
        plsc.subcore_barrier()
        pltpu.sync_copy(acc_sh.at[pl.ds(r0, rpt)],
                        part_hbm.at[cid, pl.ds(r0, rpt)])

    return edge_pass


def _deg_mm_body(dp_ref, x_ref, w_ref, g_ref, dinv_ref):
    deg = dp_ref[0, :, 0:1] + dp_ref[1, :, 0:1] + 1.0
    dinv = lax.rsqrt(deg)
    h = jnp.dot(x_ref[...], w_ref[...], preferred_element_type=jnp.float32)
    g_ref[...] = h * dinv
    dinv_ref[...] = dinv


def _mid_body(p_ref, g_ref, dinv_ref, b_ref, w_ref, out_ref):
    dinv = dinv_ref[...]
    h = jnp.maximum((p_ref[0] + p_ref[1] + g_ref[...]) * dinv + b_ref[...], 0.0)
    out_ref[...] = jnp.dot(
        h, w_ref[...], preferred_element_type=jnp.float32) * dinv


def _fin_mm_body(p_ref, g_ref, dinv_ref, b_ref, wf_ref, g3w_ref):
    dinv = dinv_ref[...]
    h = jnp.maximum((p_ref[0] + p_ref[1] + g_ref[...]) * dinv + b_ref[...], 0.0)
    g3 = jnp.dot(h, wf_ref[...], preferred_element_type=jnp.float32) * dinv
    g3w_ref[...] = jnp.broadcast_to(g3, g3w_ref.shape)


def _out_body(p_ref, g3w_ref, dinv_ref, bf_ref, out_ref):
    agg = p_ref[0, :, 0:1] + p_ref[1, :, 0:1] + g3w_ref[:, 0:1]
    out_ref[...] = agg * dinv_ref[...] + bf_ref[...]


def _tc_call(body, out_shapes):
    return pl.pallas_call(body, out_shape=out_shapes)


def kernel(x, edge_index, batch, W0, b0, W1, b1, Wf, bf):
    n, cin = x.shape
    e = edge_index.shape[1]
    c = W0.shape[1]
    k = 80
    nch = e // (NW * k)
    assert nch * NW * k == e and n % NS == 0

    src = edge_index[0].reshape(NW, nch, k)
    dst = edge_index[1].reshape(NW, nch, k)

    pass_c = _make_edge_pass(n, c, nch, k)
    pass_16 = _make_edge_pass(n, 16, nch, k)

    ones16 = jnp.ones((n, 16), jnp.float32)
    z16 = jnp.zeros((n, 16), jnp.float32)
    zc = jnp.zeros((n, c), jnp.float32)

    dp = pass_16(ones16, src, dst, z16)
    g1, dinv = _tc_call(
        _deg_mm_body,
        (jax.ShapeDtypeStruct((n, c), jnp.float32),
         jax.ShapeDtypeStruct((n, 1), jnp.float32)))(dp, x, W0)

    p1 = pass_c(g1, src, dst, zc)
    g2 = _tc_call(
        _mid_body, jax.ShapeDtypeStruct((n, c), jnp.float32))(
            p1, g1, dinv, b0.reshape(1, c), W1)

    p2 = pass_c(g2, src, dst, zc)
    g3w = _tc_call(
        _fin_mm_body, jax.ShapeDtypeStruct((n, 16), jnp.float32))(
            p2, g2, dinv, b1.reshape(1, c), Wf)

    p3 = pass_16(g3w, src, dst, z16)
    out = _tc_call(
        _out_body, jax.ShapeDtypeStruct((n, 1), jnp.float32))(
            p3, g3w, dinv, bf.reshape(1, 1))
    return out
